# Initial kernel scaffold; baseline (speedup 1.0000x reference)
import jax
import jax.numpy as jnp
from jax.experimental import pallas as pl


def kernel(position_matrix, channel_matrix, W_kernel, b_kernel, W_shortcut, b_shortcut):
    raise NotImplementedError("write your pallas kernel here")


# speedup vs baseline: 1.2803x; 1.2803x over previous
import functools

import jax
import jax.numpy as jnp
import numpy as np
from jax import lax
from jax.experimental import pallas as pl
from jax.experimental.pallas import tpu as pltpu
from jax.experimental.pallas import tpu_sc as plsc

_K = 16
_NPOLY = 4
_LANG = 6
_MRAD = 8
_RMAX = 3.0
_NB = _NPOLY + _LANG + _MRAD
_MU = np.linspace(0.0, _RMAX, _MRAD).astype(np.float32)
_SIGMA = _RMAX / _MRAD

_BQ = 80
_BN = 200


def _knn_body(posq_ref, posct_ref, idx_ref):
    q = posq_ref[...]
    ct = posct_ref[...]
    sqq = jnp.sum(q * q, axis=1, keepdims=True)
    sqc = jnp.sum(ct * ct, axis=0, keepdims=True)
    prod = jnp.dot(q.astype(jnp.bfloat16), ct.astype(jnp.bfloat16),
                   preferred_element_type=jnp.float32)
    d2 = sqq + sqc - 2.0 * prod
    cols = lax.broadcasted_iota(jnp.int32, d2.shape, 1)
    for k in range(_K):
        a = jnp.argmin(d2, axis=1).astype(jnp.int32)
        idx_ref[:, k] = a
        d2 = jnp.where(cols == a[:, None], jnp.inf, d2)


def _bf(x):
    return x.astype(jnp.bfloat16).astype(jnp.float32)


def _cov_body(gpos_ref, posq_ref, cov_ref):
    gp = gpos_ref[...]
    cp = posq_ref[...]
    rel = [_bf(gp[:, :, i] - cp[:, i:i + 1]) for i in range(3)]
    pieces = []
    for i in range(3):
        for j in range(3):
            pieces.append(jnp.sum(rel[i] * rel[j], axis=1, keepdims=True)
                          * (1.0 / _K))
    pieces.append(jnp.zeros((gp.shape[0], 7), jnp.float32))
    cov_ref[...] = jnp.concatenate(pieces, axis=1)


def _main_body(gch_ref, gpos_ref, posq_ref, v9_ref, x_ref,
               wk_ref, bk_ref, ws_ref, bs_ref, out_ref):
    gch = _bf(gch_ref[...])
    gp = gpos_ref[...]
    cp = posq_ref[...]
    v9 = v9_ref[...]
    rel = [_bf(gp[:, :, i] - cp[:, i:i + 1]) for i in range(3)]
    z = []
    for j in range(3):
        z.append(rel[0] * _bf(v9[:, j:j + 1])
                 + rel[1] * _bf(v9[:, 3 + j:4 + j])
                 + rel[2] * _bf(v9[:, 6 + j:7 + j]))
    d = jnp.sqrt(z[0] * z[0] + z[1] * z[1] + z[2] * z[2])
    dn = jnp.clip(d / _RMAX, 0.0, 1.0)
    c = z[2] / (d + 1e-8)
    dn2 = dn * dn
    c2 = c * c
    c4 = c2 * c2
    basis = [dn, dn2, dn2 * dn, dn2 * dn2,
             jnp.ones_like(c), c, c2, c2 * c, c4, c4 * c]
    two_sig2 = 2.0 * _SIGMA * _SIGMA
    for m in range(_MRAD):
        t = d - float(_MU[m])
        basis.append(jnp.exp(-(t * t) / two_sig2))
    acc = []
    for b in range(_NB):
        s = jnp.sum(_bf(basis[b])[:, :, None] * gch, axis=1)
        acc.append(s * (1.0 / _K))
    agg = jnp.concatenate(acc, axis=1)
    conv = jnp.dot(agg.astype(jnp.bfloat16), wk_ref[...],
                   preferred_element_type=jnp.float32)
    conv = conv + bk_ref[...]
    short = jnp.dot(x_ref[...].astype(jnp.bfloat16), ws_ref[...],
                    preferred_element_type=jnp.float32)
    short = short + bs_ref[...]
    out_ref[...] = jnp.maximum(conv + short, 0.0)


def _sc_gather(tab_ch, tab_pos, idx_flat, n_edges):
    info = plsc.get_sparse_core_info()
    nw = info.num_cores * info.num_subcores
    per_w = n_edges // nw
    chunk = 200
    n_chunks = per_w // chunk
    mesh = plsc.VectorSubcoreMesh(core_axis_name="c", subcore_axis_name="s")

    @functools.partial(
        pl.kernel, mesh=mesh,
        out_type=[jax.ShapeDtypeStruct((n_edges, 128), jnp.float32),
                  jax.ShapeDtypeStruct((n_edges, 128), jnp.float32)],
        scratch_types=[pltpu.VMEM((chunk,), jnp.int32),
                       pltpu.VMEM((chunk, 128), jnp.float32),
                       pltpu.VMEM((chunk, 128), jnp.float32),
                       pltpu.SemaphoreType.DMA,
                       pltpu.SemaphoreType.DMA],
    )
    def gather(ch_hbm, pos_hbm, idx_hbm, out_ch, out_pos,
               idx_v, ch_v, pos_v, sem1, sem2):
        wid = lax.axis_index("s") * info.num_cores + lax.axis_index("c")
        for ci in range(n_chunks):
            base = wid * per_w + ci * chunk
            pltpu.sync_copy(idx_hbm.at[pl.ds(base, chunk)], idx_v)
            cp1 = pltpu.async_copy(ch_hbm.at[idx_v], ch_v, sem1)
            cp2 = pltpu.async_copy(pos_hbm.at[idx_v], pos_v, sem2)
            cp1.wait()
            cp2.wait()
            pltpu.sync_copy(ch_v, out_ch.at[pl.ds(base, chunk)])
            pltpu.sync_copy(pos_v, out_pos.at[pl.ds(base, chunk)])

    return gather(tab_ch, tab_pos, idx_flat)


def kernel(position_matrix, channel_matrix, W_kernel, b_kernel,
           W_shortcut, b_shortcut):
    n = position_matrix.shape[0]
    cin = channel_matrix.shape[1]
    cout = W_shortcut.shape[1]
    npad = ((n + 1023) // 1024) * 1024

    posq = jnp.pad(position_matrix, ((0, 0), (0, 5)))
    posc = jnp.pad(posq, ((0, npad - n), (0, 0)),
                   constant_values=1e6)
    posct = posc.T

    idx = pl.pallas_call(
        _knn_body,
        grid=(n // _BQ,),
        in_specs=[pl.BlockSpec((_BQ, 8), lambda i: (i, 0)),
                  pl.BlockSpec((8, npad), lambda i: (0, 0))],
        out_specs=pl.BlockSpec((_BQ, _K), lambda i: (i, 0)),
        out_shape=jax.ShapeDtypeStruct((n, _K), jnp.int32),
    )(posq, posct)

    tab_pos = jnp.pad(position_matrix, ((0, 0), (0, 125)))
    gch, gpos = _sc_gather(channel_matrix, tab_pos,
                           idx.reshape(-1), n * _K)
    gch = gch.reshape(n, _K, cin)
    gpos = gpos.reshape(n, _K, 128)

    cov16 = pl.pallas_call(
        _cov_body,
        grid=(n // _BN,),
        in_specs=[pl.BlockSpec((_BN, _K, 128), lambda i: (i, 0, 0)),
                  pl.BlockSpec((_BN, 8), lambda i: (i, 0))],
        out_specs=pl.BlockSpec((_BN, 16), lambda i: (i, 0)),
        out_shape=jax.ShapeDtypeStruct((n, 16), jnp.float32),
    )(gpos, posq)
    cov = cov16[:, :9].reshape(n, 3, 3)

    _, V = jnp.linalg.eigh(cov)
    v9 = jnp.pad(lax.stop_gradient(V).reshape(n, 9), ((0, 0), (0, 7)))

    out = pl.pallas_call(
        _main_body,
        grid=(n // _BN,),
        in_specs=[pl.BlockSpec((_BN, _K, cin), lambda i: (i, 0, 0)),
                  pl.BlockSpec((_BN, _K, 128), lambda i: (i, 0, 0)),
                  pl.BlockSpec((_BN, 8), lambda i: (i, 0)),
                  pl.BlockSpec((_BN, 16), lambda i: (i, 0)),
                  pl.BlockSpec((_BN, cin), lambda i: (i, 0)),
                  pl.BlockSpec((_NB * cin, cout), lambda i: (0, 0)),
                  pl.BlockSpec((1, cout), lambda i: (0, 0)),
                  pl.BlockSpec((cin, cout), lambda i: (0, 0)),
                  pl.BlockSpec((1, cout), lambda i: (0, 0))],
        out_specs=pl.BlockSpec((_BN, cout), lambda i: (i, 0)),
        out_shape=jax.ShapeDtypeStruct((n, cout), jnp.float32),
    )(gch, gpos, posq, v9, channel_matrix,
      W_kernel.astype(jnp.bfloat16), b_kernel.reshape(1, cout),
      W_shortcut.astype(jnp.bfloat16), b_shortcut.reshape(1, cout))
    return out


# speedup vs baseline: 14.3142x; 11.1804x over previous
import functools

import jax
import jax.numpy as jnp
import numpy as np
from jax import lax
from jax.experimental import pallas as pl
from jax.experimental.pallas import tpu as pltpu
from jax.experimental.pallas import tpu_sc as plsc

_K = 16
_NPOLY = 4
_LANG = 6
_MRAD = 8
_RMAX = 3.0
_NB = _NPOLY + _LANG + _MRAD
_MU = np.linspace(0.0, _RMAX, _MRAD).astype(np.float32)
_SIGMA = _RMAX / _MRAD

_BQ = 80
_BN = 200


def _knn_body(posq_ref, posct_ref, idx_ref):
    q = posq_ref[...]
    ct = posct_ref[...]
    sqq = jnp.sum(q * q, axis=1, keepdims=True)
    sqc = jnp.sum(ct * ct, axis=0, keepdims=True)
    prod = jnp.dot(q.astype(jnp.bfloat16), ct.astype(jnp.bfloat16),
                   preferred_element_type=jnp.float32)
    d2 = sqq + sqc - 2.0 * prod
    cols = lax.broadcasted_iota(jnp.int32, d2.shape, 1)
    for k in range(_K):
        a = jnp.argmin(d2, axis=1).astype(jnp.int32)
        idx_ref[:, k] = a
        d2 = jnp.where(cols == a[:, None], jnp.inf, d2)


def _bf(x):
    return x.astype(jnp.bfloat16).astype(jnp.float32)


def _cov_body(gpos_ref, posq_ref, cov_ref):
    gp = gpos_ref[...]
    cp = posq_ref[...]
    rel = [_bf(gp[:, :, i] - cp[:, i:i + 1]) for i in range(3)]
    pieces = []
    for i in range(3):
        for j in range(3):
            pieces.append(jnp.sum(rel[i] * rel[j], axis=1, keepdims=True)
                          * (1.0 / _K))
    pieces.append(jnp.zeros((gp.shape[0], 7), jnp.float32))
    cov_ref[...] = jnp.concatenate(pieces, axis=1)


def _jacobi_body(cov_ref, v_ref):
    cv = cov_ref[...]
    blk = cv.shape[1]
    A = [[cv[0:1, :], cv[1:2, :], cv[2:3, :]],
         [cv[1:2, :], cv[3:4, :], cv[4:5, :]],
         [cv[2:3, :], cv[4:5, :], cv[5:6, :]]]
    one = jnp.ones((1, blk), jnp.float32)
    zero = jnp.zeros((1, blk), jnp.float32)
    V = [[one, zero, zero], [zero, one, zero], [zero, zero, one]]
    for _ in range(14):
        for (p, q) in ((0, 2), (1, 2), (0, 1)):
            apq = A[p][q]
            nz = apq != 0.0
            tau = jnp.where(nz, (A[q][q] - A[p][p]) / (2.0 * apq), 0.0)
            t = jnp.where(nz, jnp.sign(tau)
                          / (jnp.abs(tau) + jnp.sqrt(1.0 + tau * tau)), 0.0)
            c = 1.0 / jnp.sqrt(1.0 + t * t)
            s = t * c
            for j in range(3):
                rp = c * A[p][j] - s * A[q][j]
                rq = s * A[p][j] + c * A[q][j]
                A[p][j] = rp
                A[q][j] = rq
            for i in range(3):
                cp_ = c * A[i][p] - s * A[i][q]
                cq_ = s * A[i][p] + c * A[i][q]
                A[i][p] = cp_
                A[i][q] = cq_
                vp = c * V[i][p] - s * V[i][q]
                vq = s * V[i][p] + c * V[i][q]
                V[i][p] = vp
                V[i][q] = vq
    w0, w1, w2 = A[0][0], A[1][1], A[2][2]
    is2 = jnp.logical_and(w2 >= w0, w2 >= w1)
    is1 = jnp.logical_and(jnp.logical_not(is2), w1 >= w0)
    rows = []
    for i in range(3):
        main = jnp.where(is2, V[i][2], jnp.where(is1, V[i][1], V[i][0]))
        resta = jnp.where(jnp.logical_or(is2, is1), V[i][0], V[i][1])
        restb = jnp.where(is2, V[i][1], V[i][2])
        rows.extend([resta, restb, main])
    rows.extend([zero] * 7)
    v_ref[...] = jnp.concatenate(rows, axis=0)


def _main_body(gch_ref, gpos_ref, posq_ref, v9_ref, x_ref,
               wk_ref, bk_ref, ws_ref, bs_ref, out_ref):
    gch = _bf(gch_ref[...])
    gp = gpos_ref[...]
    cp = posq_ref[...]
    v9 = v9_ref[...]
    rel = [_bf(gp[:, :, i] - cp[:, i:i + 1]) for i in range(3)]
    z = []
    for j in range(3):
        z.append(rel[0] * _bf(v9[:, j:j + 1])
                 + rel[1] * _bf(v9[:, 3 + j:4 + j])
                 + rel[2] * _bf(v9[:, 6 + j:7 + j]))
    d = jnp.sqrt(z[0] * z[0] + z[1] * z[1] + z[2] * z[2])
    dn = jnp.clip(d / _RMAX, 0.0, 1.0)
    c = z[2] / (d + 1e-8)
    dn2 = dn * dn
    c2 = c * c
    c4 = c2 * c2
    basis = [dn, dn2, dn2 * dn, dn2 * dn2,
             jnp.ones_like(c), c, c2, c2 * c, c4, c4 * c]
    two_sig2 = 2.0 * _SIGMA * _SIGMA
    for m in range(_MRAD):
        t = d - float(_MU[m])
        basis.append(jnp.exp(-(t * t) / two_sig2))
    acc = []
    for b in range(_NB):
        s = jnp.sum(_bf(basis[b])[:, :, None] * gch, axis=1)
        acc.append(s * (1.0 / _K))
    agg = jnp.concatenate(acc, axis=1)
    conv = jnp.dot(agg.astype(jnp.bfloat16), wk_ref[...],
                   preferred_element_type=jnp.float32)
    conv = conv + bk_ref[...]
    short = jnp.dot(x_ref[...].astype(jnp.bfloat16), ws_ref[...],
                    preferred_element_type=jnp.float32)
    short = short + bs_ref[...]
    out_ref[...] = jnp.maximum(conv + short, 0.0)


def _sc_gather(tab_ch, tab_pos, idx_flat, n_edges):
    info = plsc.get_sparse_core_info()
    nw = info.num_cores * info.num_subcores
    per_w = n_edges // nw
    chunk = 200
    n_chunks = per_w // chunk
    mesh = plsc.VectorSubcoreMesh(core_axis_name="c", subcore_axis_name="s")

    @functools.partial(
        pl.kernel, mesh=mesh,
        out_type=[jax.ShapeDtypeStruct((n_edges, 128), jnp.float32),
                  jax.ShapeDtypeStruct((n_edges, 128), jnp.float32)],
        scratch_types=[pltpu.VMEM((chunk,), jnp.int32),
                       pltpu.VMEM((chunk, 128), jnp.float32),
                       pltpu.VMEM((chunk, 128), jnp.float32),
                       pltpu.SemaphoreType.DMA,
                       pltpu.SemaphoreType.DMA],
    )
    def gather(ch_hbm, pos_hbm, idx_hbm, out_ch, out_pos,
               idx_v, ch_v, pos_v, sem1, sem2):
        wid = lax.axis_index("s") * info.num_cores + lax.axis_index("c")
        for ci in range(n_chunks):
            base = wid * per_w + ci * chunk
            pltpu.sync_copy(idx_hbm.at[pl.ds(base, chunk)], idx_v)
            cp1 = pltpu.async_copy(ch_hbm.at[idx_v], ch_v, sem1)
            cp2 = pltpu.async_copy(pos_hbm.at[idx_v], pos_v, sem2)
            cp1.wait()
            cp2.wait()
            pltpu.sync_copy(ch_v, out_ch.at[pl.ds(base, chunk)])
            pltpu.sync_copy(pos_v, out_pos.at[pl.ds(base, chunk)])

    return gather(tab_ch, tab_pos, idx_flat)


def kernel(position_matrix, channel_matrix, W_kernel, b_kernel,
           W_shortcut, b_shortcut):
    n = position_matrix.shape[0]
    cin = channel_matrix.shape[1]
    cout = W_shortcut.shape[1]
    npad = ((n + 1023) // 1024) * 1024

    posq = jnp.pad(position_matrix, ((0, 0), (0, 5)))
    posc = jnp.pad(posq, ((0, npad - n), (0, 0)),
                   constant_values=1e6)
    posct = posc.T

    idx = pl.pallas_call(
        _knn_body,
        grid=(n // _BQ,),
        in_specs=[pl.BlockSpec((_BQ, 8), lambda i: (i, 0)),
                  pl.BlockSpec((8, npad), lambda i: (0, 0))],
        out_specs=pl.BlockSpec((_BQ, _K), lambda i: (i, 0)),
        out_shape=jax.ShapeDtypeStruct((n, _K), jnp.int32),
    )(posq, posct)

    tab_pos = jnp.pad(position_matrix, ((0, 0), (0, 125)))
    gch, gpos = _sc_gather(channel_matrix, tab_pos,
                           idx.reshape(-1), n * _K)
    gch = gch.reshape(n, _K, cin)
    gpos = gpos.reshape(n, _K, 128)

    cov16 = pl.pallas_call(
        _cov_body,
        grid=(n // _BN,),
        in_specs=[pl.BlockSpec((_BN, _K, 128), lambda i: (i, 0, 0)),
                  pl.BlockSpec((_BN, 8), lambda i: (i, 0))],
        out_specs=pl.BlockSpec((_BN, 16), lambda i: (i, 0)),
        out_shape=jax.ShapeDtypeStruct((n, 16), jnp.float32),
    )(gpos, posq)
    cov6t = jnp.pad(cov16[:, (0, 1, 2, 4, 5, 8)].T, ((0, 2), (0, 0)))
    v9t = pl.pallas_call(
        _jacobi_body,
        grid=(1,),
        in_specs=[pl.BlockSpec((8, n), lambda i: (0, 0))],
        out_specs=pl.BlockSpec((16, n), lambda i: (0, 0)),
        out_shape=jax.ShapeDtypeStruct((16, n), jnp.float32),
    )(cov6t)
    v9 = jnp.pad(v9t[:9].T, ((0, 0), (0, 7)))

    out = pl.pallas_call(
        _main_body,
        grid=(n // _BN,),
        in_specs=[pl.BlockSpec((_BN, _K, cin), lambda i: (i, 0, 0)),
                  pl.BlockSpec((_BN, _K, 128), lambda i: (i, 0, 0)),
                  pl.BlockSpec((_BN, 8), lambda i: (i, 0)),
                  pl.BlockSpec((_BN, 16), lambda i: (i, 0)),
                  pl.BlockSpec((_BN, cin), lambda i: (i, 0)),
                  pl.BlockSpec((_NB * cin, cout), lambda i: (0, 0)),
                  pl.BlockSpec((1, cout), lambda i: (0, 0)),
                  pl.BlockSpec((cin, cout), lambda i: (0, 0)),
                  pl.BlockSpec((1, cout), lambda i: (0, 0))],
        out_specs=pl.BlockSpec((_BN, cout), lambda i: (i, 0)),
        out_shape=jax.ShapeDtypeStruct((n, cout), jnp.float32),
    )(gch, gpos, posq, v9, channel_matrix,
      W_kernel.astype(jnp.bfloat16), b_kernel.reshape(1, cout),
      W_shortcut.astype(jnp.bfloat16), b_shortcut.reshape(1, cout))
    return out


# speedup vs baseline: 14.7435x; 1.0300x over previous
import functools

import jax
import jax.numpy as jnp
import numpy as np
from jax import lax
from jax.experimental import pallas as pl
from jax.experimental.pallas import tpu as pltpu
from jax.experimental.pallas import tpu_sc as plsc

_K = 16
_NPOLY = 4
_LANG = 6
_MRAD = 8
_RMAX = 3.0
_NB = _NPOLY + _LANG + _MRAD
_MU = np.linspace(0.0, _RMAX, _MRAD).astype(np.float32)
_SIGMA = _RMAX / _MRAD

_BQ = 200
_BN = 200


def _knn_body(posq_ref, posct_ref, idx_ref):
    q = posq_ref[...]
    ct = posct_ref[...]
    sqq = jnp.sum(q * q, axis=1, keepdims=True)
    sqc = jnp.sum(ct * ct, axis=0, keepdims=True)
    prod = jnp.dot(q.astype(jnp.bfloat16), ct.astype(jnp.bfloat16),
                   preferred_element_type=jnp.float32)
    d2 = sqq + sqc - 2.0 * prod
    cols = lax.broadcasted_iota(jnp.int32, d2.shape, 1)
    for k in range(_K):
        a = jnp.argmin(d2, axis=1).astype(jnp.int32)
        idx_ref[:, k] = a
        d2 = jnp.where(cols == a[:, None], jnp.inf, d2)


def _bf(x):
    return x.astype(jnp.bfloat16).astype(jnp.float32)


def _cov_body(gpos_ref, posq_ref, cov_ref):
    gp = gpos_ref[...]
    cp = posq_ref[...]
    rel = [_bf(gp[:, :, i] - cp[:, i:i + 1]) for i in range(3)]
    pieces = []
    for i in range(3):
        for j in range(3):
            pieces.append(jnp.sum(rel[i] * rel[j], axis=1, keepdims=True)
                          * (1.0 / _K))
    pieces.append(jnp.zeros((gp.shape[0], 7), jnp.float32))
    cov_ref[...] = jnp.concatenate(pieces, axis=1)


def _jacobi_body(cov_ref, v_ref):
    cv = cov_ref[...]
    blk = cv.shape[1]
    A = [[cv[0:1, :], cv[1:2, :], cv[2:3, :]],
         [cv[1:2, :], cv[3:4, :], cv[4:5, :]],
         [cv[2:3, :], cv[4:5, :], cv[5:6, :]]]
    one = jnp.ones((1, blk), jnp.float32)
    zero = jnp.zeros((1, blk), jnp.float32)
    V = [[one, zero, zero], [zero, one, zero], [zero, zero, one]]
    for _ in range(14):
        for (p, q) in ((0, 2), (1, 2), (0, 1)):
            apq = A[p][q]
            nz = apq != 0.0
            tau = jnp.where(nz, (A[q][q] - A[p][p]) / (2.0 * apq), 0.0)
            t = jnp.where(nz, jnp.sign(tau)
                          / (jnp.abs(tau) + jnp.sqrt(1.0 + tau * tau)), 0.0)
            c = 1.0 / jnp.sqrt(1.0 + t * t)
            s = t * c
            for j in range(3):
                rp = c * A[p][j] - s * A[q][j]
                rq = s * A[p][j] + c * A[q][j]
                A[p][j] = rp
                A[q][j] = rq
            for i in range(3):
                cp_ = c * A[i][p] - s * A[i][q]
                cq_ = s * A[i][p] + c * A[i][q]
                A[i][p] = cp_
                A[i][q] = cq_
                vp = c * V[i][p] - s * V[i][q]
                vq = s * V[i][p] + c * V[i][q]
                V[i][p] = vp
                V[i][q] = vq
    w0, w1, w2 = A[0][0], A[1][1], A[2][2]
    is2 = jnp.logical_and(w2 >= w0, w2 >= w1)
    is1 = jnp.logical_and(jnp.logical_not(is2), w1 >= w0)
    rows = []
    for i in range(3):
        main = jnp.where(is2, V[i][2], jnp.where(is1, V[i][1], V[i][0]))
        resta = jnp.where(jnp.logical_or(is2, is1), V[i][0], V[i][1])
        restb = jnp.where(is2, V[i][1], V[i][2])
        rows.extend([resta, restb, main])
    rows.extend([zero] * 7)
    v_ref[...] = jnp.concatenate(rows, axis=0)


def _main_body(gch_ref, gpos_ref, posq_ref, v9_ref, x_ref,
               wk_ref, bk_ref, ws_ref, bs_ref, out_ref):
    gch = _bf(gch_ref[...])
    gp = gpos_ref[...]
    cp = posq_ref[...]
    v9 = v9_ref[...]
    rel = [_bf(gp[:, :, i] - cp[:, i:i + 1]) for i in range(3)]
    z = []
    for j in range(3):
        z.append(rel[0] * _bf(v9[:, j:j + 1])
                 + rel[1] * _bf(v9[:, 3 + j:4 + j])
                 + rel[2] * _bf(v9[:, 6 + j:7 + j]))
    d = jnp.sqrt(z[0] * z[0] + z[1] * z[1] + z[2] * z[2])
    dn = jnp.clip(d / _RMAX, 0.0, 1.0)
    c = z[2] / (d + 1e-8)
    dn2 = dn * dn
    c2 = c * c
    c4 = c2 * c2
    basis = [dn, dn2, dn2 * dn, dn2 * dn2,
             jnp.ones_like(c), c, c2, c2 * c, c4, c4 * c]
    two_sig2 = 2.0 * _SIGMA * _SIGMA
    for m in range(_MRAD):
        t = d - float(_MU[m])
        basis.append(jnp.exp(-(t * t) / two_sig2))
    acc = []
    for b in range(_NB):
        s = jnp.sum(_bf(basis[b])[:, :, None] * gch, axis=1)
        acc.append(s * (1.0 / _K))
    agg = jnp.concatenate(acc, axis=1)
    conv = jnp.dot(agg.astype(jnp.bfloat16), wk_ref[...],
                   preferred_element_type=jnp.float32)
    conv = conv + bk_ref[...]
    short = jnp.dot(x_ref[...].astype(jnp.bfloat16), ws_ref[...],
                    preferred_element_type=jnp.float32)
    short = short + bs_ref[...]
    out_ref[...] = jnp.maximum(conv + short, 0.0)


def _sc_gather(tab_ch, tab_pos, idx_flat, n_edges):
    info = plsc.get_sparse_core_info()
    nw = info.num_cores * info.num_subcores
    per_w = n_edges // nw
    chunk = 200
    n_chunks = per_w // chunk
    mesh = plsc.VectorSubcoreMesh(core_axis_name="c", subcore_axis_name="s")

    @functools.partial(
        pl.kernel, mesh=mesh,
        out_type=[jax.ShapeDtypeStruct((n_edges, 128), jnp.float32),
                  jax.ShapeDtypeStruct((n_edges, 128), jnp.float32)],
        scratch_types=[pltpu.VMEM((chunk,), jnp.int32),
                       pltpu.VMEM((chunk, 128), jnp.float32),
                       pltpu.VMEM((chunk, 128), jnp.float32),
                       pltpu.SemaphoreType.DMA,
                       pltpu.SemaphoreType.DMA],
    )
    def gather(ch_hbm, pos_hbm, idx_hbm, out_ch, out_pos,
               idx_v, ch_v, pos_v, sem1, sem2):
        wid = lax.axis_index("s") * info.num_cores + lax.axis_index("c")
        for ci in range(n_chunks):
            base = wid * per_w + ci * chunk
            pltpu.sync_copy(idx_hbm.at[pl.ds(base, chunk)], idx_v)
            cp1 = pltpu.async_copy(ch_hbm.at[idx_v], ch_v, sem1)
            cp2 = pltpu.async_copy(pos_hbm.at[idx_v], pos_v, sem2)
            cp1.wait()
            cp2.wait()
            pltpu.sync_copy(ch_v, out_ch.at[pl.ds(base, chunk)])
            pltpu.sync_copy(pos_v, out_pos.at[pl.ds(base, chunk)])

    return gather(tab_ch, tab_pos, idx_flat)


def kernel(position_matrix, channel_matrix, W_kernel, b_kernel,
           W_shortcut, b_shortcut):
    n = position_matrix.shape[0]
    cin = channel_matrix.shape[1]
    cout = W_shortcut.shape[1]
    npad = ((n + 1023) // 1024) * 1024

    posq = jnp.pad(position_matrix, ((0, 0), (0, 5)))
    posc = jnp.pad(posq, ((0, npad - n), (0, 0)),
                   constant_values=1e6)
    posct = posc.T

    idx = pl.pallas_call(
        _knn_body,
        grid=(n // _BQ,),
        in_specs=[pl.BlockSpec((_BQ, 8), lambda i: (i, 0)),
                  pl.BlockSpec((8, npad), lambda i: (0, 0))],
        out_specs=pl.BlockSpec((_BQ, _K), lambda i: (i, 0)),
        out_shape=jax.ShapeDtypeStruct((n, _K), jnp.int32),
    )(posq, posct)

    tab_pos = jnp.pad(position_matrix, ((0, 0), (0, 125)))
    gch, gpos = _sc_gather(channel_matrix, tab_pos,
                           idx.reshape(-1), n * _K)
    gch = gch.reshape(n, _K, cin)
    gpos = gpos.reshape(n, _K, 128)

    cov16 = pl.pallas_call(
        _cov_body,
        grid=(n // _BN,),
        in_specs=[pl.BlockSpec((_BN, _K, 128), lambda i: (i, 0, 0)),
                  pl.BlockSpec((_BN, 8), lambda i: (i, 0))],
        out_specs=pl.BlockSpec((_BN, 16), lambda i: (i, 0)),
        out_shape=jax.ShapeDtypeStruct((n, 16), jnp.float32),
    )(gpos, posq)
    cov6t = jnp.pad(cov16[:, (0, 1, 2, 4, 5, 8)].T, ((0, 2), (0, 0)))
    v9t = pl.pallas_call(
        _jacobi_body,
        grid=(1,),
        in_specs=[pl.BlockSpec((8, n), lambda i: (0, 0))],
        out_specs=pl.BlockSpec((16, n), lambda i: (0, 0)),
        out_shape=jax.ShapeDtypeStruct((16, n), jnp.float32),
    )(cov6t)
    v9 = jnp.pad(v9t[:9].T, ((0, 0), (0, 7)))

    out = pl.pallas_call(
        _main_body,
        grid=(n // _BN,),
        in_specs=[pl.BlockSpec((_BN, _K, cin), lambda i: (i, 0, 0)),
                  pl.BlockSpec((_BN, _K, 128), lambda i: (i, 0, 0)),
                  pl.BlockSpec((_BN, 8), lambda i: (i, 0)),
                  pl.BlockSpec((_BN, 16), lambda i: (i, 0)),
                  pl.BlockSpec((_BN, cin), lambda i: (i, 0)),
                  pl.BlockSpec((_NB * cin, cout), lambda i: (0, 0)),
                  pl.BlockSpec((1, cout), lambda i: (0, 0)),
                  pl.BlockSpec((cin, cout), lambda i: (0, 0)),
                  pl.BlockSpec((1, cout), lambda i: (0, 0))],
        out_specs=pl.BlockSpec((_BN, cout), lambda i: (i, 0)),
        out_shape=jax.ShapeDtypeStruct((n, cout), jnp.float32),
    )(gch, gpos, posq, v9, channel_matrix,
      W_kernel.astype(jnp.bfloat16), b_kernel.reshape(1, cout),
      W_shortcut.astype(jnp.bfloat16), b_shortcut.reshape(1, cout))
    return out
